# baseline (speedup 1.0000x reference)
import jax
import jax.numpy as jnp
from jax.experimental import pallas as pl
from jax.experimental.pallas import tpu as pltpu

_CHUNK = 256
_NCHUNK_PER_ARR = 8
_NCHUNK = 2 * _NCHUNK_PER_ARR


def _stream_kernel(u_hbm, i_hbm, uo_hbm, io_hbm, buf, sin, sout):
    pairs = [(u_hbm, uo_hbm), (i_hbm, io_hbm)]
    ins = []
    for k in range(_NCHUNK):
        src, dst = pairs[k // _NCHUNK_PER_ARR]
        r0 = (k % _NCHUNK_PER_ARR) * _CHUNK
        cin = pltpu.make_async_copy(src.at[pl.ds(r0, _CHUNK)], buf.at[k],
                                    sin.at[k])
        cin.start()
        ins.append((cin, dst, r0))
    outs = []
    for k, (cin, dst, r0) in enumerate(ins):
        cin.wait()
        cout = pltpu.make_async_copy(buf.at[k], dst.at[pl.ds(r0, _CHUNK)],
                                     sout.at[k])
        cout.start()
        outs.append(cout)
    for cout in outs:
        cout.wait()


def kernel(user_emb, item_emb, adj_rows, adj_cols, norm_vals, adj_vals,
           w_q, b_q, w_k, b_k, w_v, b_v):
    n_user, emb = user_emb.shape
    user_out, item_out = pl.pallas_call(
        _stream_kernel,
        in_specs=[pl.BlockSpec(memory_space=pl.ANY),
                  pl.BlockSpec(memory_space=pl.ANY)],
        out_specs=[pl.BlockSpec(memory_space=pl.ANY),
                   pl.BlockSpec(memory_space=pl.ANY)],
        out_shape=[
            jax.ShapeDtypeStruct(user_emb.shape, user_emb.dtype),
            jax.ShapeDtypeStruct(item_emb.shape, item_emb.dtype),
        ],
        scratch_shapes=[
            pltpu.VMEM((_NCHUNK, _CHUNK, emb), user_emb.dtype),
            pltpu.SemaphoreType.DMA((_NCHUNK,)),
            pltpu.SemaphoreType.DMA((_NCHUNK,)),
        ],
    )(user_emb, item_emb)
    return user_out, item_out


# speedup vs baseline: 1.0476x; 1.0476x over previous
import jax
import jax.numpy as jnp
from jax.experimental import pallas as pl
from jax.experimental.pallas import tpu as pltpu

_CHUNK = 512
_NCHUNK_PER_ARR = 4
_NCHUNK = 2 * _NCHUNK_PER_ARR


def _stream_kernel(u_hbm, i_hbm, uo_hbm, io_hbm, buf, sin, sout):
    pairs = [(u_hbm, uo_hbm), (i_hbm, io_hbm)]
    ins = []
    for k in range(_NCHUNK):
        src, dst = pairs[k // _NCHUNK_PER_ARR]
        r0 = (k % _NCHUNK_PER_ARR) * _CHUNK
        cin = pltpu.make_async_copy(src.at[pl.ds(r0, _CHUNK)], buf.at[k],
                                    sin.at[k])
        cin.start()
        ins.append((cin, dst, r0))
    outs = []
    for k, (cin, dst, r0) in enumerate(ins):
        cin.wait()
        cout = pltpu.make_async_copy(buf.at[k], dst.at[pl.ds(r0, _CHUNK)],
                                     sout.at[k])
        cout.start()
        outs.append(cout)
    for cout in outs:
        cout.wait()


def kernel(user_emb, item_emb, adj_rows, adj_cols, norm_vals, adj_vals,
           w_q, b_q, w_k, b_k, w_v, b_v):
    n_user, emb = user_emb.shape
    user_out, item_out = pl.pallas_call(
        _stream_kernel,
        in_specs=[pl.BlockSpec(memory_space=pl.ANY),
                  pl.BlockSpec(memory_space=pl.ANY)],
        out_specs=[pl.BlockSpec(memory_space=pl.ANY),
                   pl.BlockSpec(memory_space=pl.ANY)],
        out_shape=[
            jax.ShapeDtypeStruct(user_emb.shape, user_emb.dtype),
            jax.ShapeDtypeStruct(item_emb.shape, item_emb.dtype),
        ],
        scratch_shapes=[
            pltpu.VMEM((_NCHUNK, _CHUNK, emb), user_emb.dtype),
            pltpu.SemaphoreType.DMA((_NCHUNK,)),
            pltpu.SemaphoreType.DMA((_NCHUNK,)),
        ],
    )(user_emb, item_emb)
    return user_out, item_out


# speedup vs baseline: 1.0616x; 1.0134x over previous
import jax
import jax.numpy as jnp
from jax.experimental import pallas as pl
from jax.experimental.pallas import tpu as pltpu

_CHUNK = 512
_NCHUNK_PER_ARR = 4
_NCHUNK = 2 * _NCHUNK_PER_ARR


def _stream_kernel(u_hbm, i_hbm, uo_hbm, io_hbm, buf, sin, sout):
    pairs = [(u_hbm, uo_hbm), (i_hbm, io_hbm)]
    ins = []
    for k in range(_NCHUNK):
        src, dst = pairs[k % 2]
        r0 = (k // 2) * _CHUNK
        cin = pltpu.make_async_copy(src.at[pl.ds(r0, _CHUNK)], buf.at[k],
                                    sin.at[k])
        cin.start()
        ins.append((cin, dst, r0))
    outs = []
    for k, (cin, dst, r0) in enumerate(ins):
        cin.wait()
        cout = pltpu.make_async_copy(buf.at[k], dst.at[pl.ds(r0, _CHUNK)],
                                     sout.at[k])
        cout.start()
        outs.append(cout)
    for cout in outs:
        cout.wait()


def kernel(user_emb, item_emb, adj_rows, adj_cols, norm_vals, adj_vals,
           w_q, b_q, w_k, b_k, w_v, b_v):
    n_user, emb = user_emb.shape
    user_out, item_out = pl.pallas_call(
        _stream_kernel,
        in_specs=[pl.BlockSpec(memory_space=pl.ANY),
                  pl.BlockSpec(memory_space=pl.ANY)],
        out_specs=[pl.BlockSpec(memory_space=pl.ANY),
                   pl.BlockSpec(memory_space=pl.ANY)],
        out_shape=[
            jax.ShapeDtypeStruct(user_emb.shape, user_emb.dtype),
            jax.ShapeDtypeStruct(item_emb.shape, item_emb.dtype),
        ],
        scratch_shapes=[
            pltpu.VMEM((_NCHUNK, _CHUNK, emb), user_emb.dtype),
            pltpu.SemaphoreType.DMA((_NCHUNK,)),
            pltpu.SemaphoreType.DMA((_NCHUNK,)),
        ],
    )(user_emb, item_emb)
    return user_out, item_out
